# baseline (speedup 1.0000x reference)
import functools

import jax
import jax.numpy as jnp
from jax import lax
from jax.experimental import pallas as pl
from jax.experimental.pallas import tpu as pltpu
from jax.experimental.pallas import tpu_sc as plsc

N_NODES = 100000
DIM = 416
NOUT = 160
NC = 2
NS = 16
NW = NC * NS
R = 40
SLAB = 50000


def _sqrt16(x):
    i = lax.bitcast_convert_type(x, jnp.int32)
    i = jnp.int32(0x5F3759DF) - lax.shift_right_arithmetic(i, 1)
    y = lax.bitcast_convert_type(i, jnp.float32)
    xh = x * jnp.float32(0.5)
    y = y * (jnp.float32(1.5) - xh * y * y)
    return x * y


def _make_norm_kernel(n_rows):
    nchunk = n_rows // R
    niter = -(-nchunk // NW)
    assert n_rows % R == 0 and niter >= 2
    mesh = plsc.VectorSubcoreMesh(core_axis_name="c", subcore_axis_name="s")

    @functools.partial(
        pl.kernel,
        mesh=mesh,
        out_type=jax.ShapeDtypeStruct((n_rows, NOUT), jnp.float32),
        compiler_params=pltpu.CompilerParams(needs_layout_passes=False),
        scratch_types=[
            pltpu.VMEM((R, DIM), jnp.float32),
            pltpu.VMEM((R, DIM), jnp.float32),
            pltpu.VMEM((R, NOUT), jnp.float32),
            pltpu.VMEM((R, NOUT), jnp.float32),
            pltpu.SemaphoreType.DMA,
            pltpu.SemaphoreType.DMA,
            pltpu.SemaphoreType.DMA,
            pltpu.SemaphoreType.DMA,
        ],
    )
    def norm_kernel(feat_hbm, out_hbm, in_a, in_b, out_a, out_b,
                    in_sem_a, in_sem_b, out_sem_a, out_sem_b):
        wid = lax.axis_index("s") * NC + lax.axis_index("c")
        lanes = lax.iota(jnp.int32, 16)
        l3 = [lanes * jnp.int32(3) + jnp.int32(k) for k in range(3)]
        l5 = [lanes * jnp.int32(5) + jnp.int32(k) for k in range(5)]

        def chunk_of(i):
            return jnp.minimum(i * NW + wid, nchunk - 1)

        def start_in(buf, sem, i):
            r0 = chunk_of(i) * R
            pltpu.async_copy(feat_hbm.at[pl.ds(r0, R), :], buf, sem)

        def wait_in(buf, sem):
            pltpu.make_async_copy(feat_hbm.at[pl.ds(0, R), :], buf, sem).wait()

        def start_out(buf, sem, i):
            r0 = chunk_of(i) * R
            pltpu.async_copy(buf, out_hbm.at[pl.ds(r0, R), :], sem)

        def wait_out(buf, sem):
            pltpu.make_async_copy(buf, out_hbm.at[pl.ds(0, R), :], sem).wait()

        g3 = [[l3[k] + jnp.int32(64 + 48 * j) for k in range(3)]
              for j in range(4)]
        g5 = [[l5[k] + jnp.int32(256 + 80 * j) for k in range(5)]
              for j in range(2)]

        def compute(in_v, out_v):
            @plsc.parallel_loop(0, R, unroll=4)
            def row_body(r):
                splat_r = lanes * jnp.int32(0) + r
                for j in range(4):
                    v = in_v[r, pl.ds(j * 16, 16)]
                    out_v[r, pl.ds(j * 16, 16)] = lax.abs(v)
                for j in range(4):
                    g = [plsc.load_gather(in_v, [splat_r, g3[j][k]])
                         for k in range(3)]
                    acc = (g[0] * g[0] + g[1] * g[1]) + g[2] * g[2]
                    out_v[r, pl.ds(64 + j * 16, 16)] = _sqrt16(acc)
                for j in range(2):
                    g = [plsc.load_gather(in_v, [splat_r, g5[j][k]])
                         for k in range(5)]
                    acc = ((g[0] * g[0] + g[1] * g[1])
                           + (g[2] * g[2] + g[3] * g[3])) + g[4] * g[4]
                    out_v[r, pl.ds(128 + j * 16, 16)] = _sqrt16(acc)

        start_in(in_a, in_sem_a, 0)
        start_in(in_b, in_sem_b, 1)
        wait_in(in_a, in_sem_a)
        compute(in_a, out_a)
        start_out(out_a, out_sem_a, 0)
        start_in(in_a, in_sem_a, 2)
        wait_in(in_b, in_sem_b)
        compute(in_b, out_b)
        start_out(out_b, out_sem_b, 1)

        def chunk_pair(h, carry):
            i0 = h * 2
            start_in(in_b, in_sem_b, i0 + 1)
            wait_in(in_a, in_sem_a)
            wait_out(out_a, out_sem_a)
            compute(in_a, out_a)
            start_out(out_a, out_sem_a, i0)
            start_in(in_a, in_sem_a, i0 + 2)
            wait_in(in_b, in_sem_b)
            wait_out(out_b, out_sem_b)
            compute(in_b, out_b)
            start_out(out_b, out_sem_b, i0 + 1)
            return carry

        if niter % 2:
            lax.fori_loop(1, (niter - 1) // 2, chunk_pair, 0)
            wait_in(in_a, in_sem_a)
            wait_out(out_a, out_sem_a)
            compute(in_a, out_a)
            start_out(out_a, out_sem_a, niter - 1)
        else:
            lax.fori_loop(1, niter // 2, chunk_pair, 0)
            wait_in(in_a, in_sem_a)
        wait_out(out_a, out_sem_a)
        wait_out(out_b, out_sem_b)

    return norm_kernel


_norm_kernel_slab = _make_norm_kernel(SLAB)


def kernel(features):
    outs = [_norm_kernel_slab(features[i:i + SLAB])
            for i in range(0, N_NODES, SLAB)]
    return jnp.concatenate(outs, axis=0)


# speedup vs baseline: 1.0915x; 1.0915x over previous
import functools

import jax
import jax.numpy as jnp
from jax import lax
from jax.experimental import pallas as pl
from jax.experimental.pallas import tpu as pltpu
from jax.experimental.pallas import tpu_sc as plsc

N_NODES = 100000
DIM = 416
NOUT = 160
NC = 2
NS = 16
NW = NC * NS
R = 40
SLAB = 50000


def _sqrt16(x):
    i = lax.bitcast_convert_type(x, jnp.int32)
    i = jnp.int32(0x5F3759DF) - lax.shift_right_arithmetic(i, 1)
    y = lax.bitcast_convert_type(i, jnp.float32)
    xh = x * jnp.float32(0.5)
    y = y * (jnp.float32(1.5) - xh * y * y)
    return x * y


def _make_norm_kernel(n_rows, row0):
    nchunk = n_rows // R
    chunk0 = row0 // R
    niter = -(-nchunk // NW)
    assert n_rows % R == 0 and row0 % R == 0 and niter >= 2
    mesh = plsc.VectorSubcoreMesh(core_axis_name="c", subcore_axis_name="s")

    @functools.partial(
        pl.kernel,
        mesh=mesh,
        out_type=jax.ShapeDtypeStruct((n_rows, NOUT), jnp.float32),
        compiler_params=pltpu.CompilerParams(needs_layout_passes=False),
        scratch_types=[
            pltpu.VMEM((R, DIM), jnp.float32),
            pltpu.VMEM((R, DIM), jnp.float32),
            pltpu.VMEM((R, NOUT), jnp.float32),
            pltpu.VMEM((R, NOUT), jnp.float32),
            pltpu.SemaphoreType.DMA,
            pltpu.SemaphoreType.DMA,
            pltpu.SemaphoreType.DMA,
            pltpu.SemaphoreType.DMA,
        ],
    )
    def norm_kernel(feat_hbm, out_hbm, in_a, in_b, out_a, out_b,
                    in_sem_a, in_sem_b, out_sem_a, out_sem_b):
        wid = lax.axis_index("s") * NC + lax.axis_index("c")
        lanes = lax.iota(jnp.int32, 16)
        l3 = [lanes * jnp.int32(3) + jnp.int32(k) for k in range(3)]
        l5 = [lanes * jnp.int32(5) + jnp.int32(k) for k in range(5)]

        def chunk_of(i):
            return jnp.minimum(i * NW + wid, nchunk - 1)

        def start_in(buf, sem, i):
            r0 = (chunk0 + chunk_of(i)) * R
            pltpu.async_copy(feat_hbm.at[pl.ds(r0, R), :], buf, sem)

        def wait_in(buf, sem):
            pltpu.make_async_copy(feat_hbm.at[pl.ds(0, R), :], buf, sem).wait()

        def start_out(buf, sem, i):
            r0 = chunk_of(i) * R
            pltpu.async_copy(buf, out_hbm.at[pl.ds(r0, R), :], sem)

        def wait_out(buf, sem):
            pltpu.make_async_copy(buf, out_hbm.at[pl.ds(0, R), :], sem).wait()

        g3 = [[l3[k] + jnp.int32(64 + 48 * j) for k in range(3)]
              for j in range(4)]
        g5 = [[l5[k] + jnp.int32(256 + 80 * j) for k in range(5)]
              for j in range(2)]

        def compute(in_v, out_v):
            @plsc.parallel_loop(0, R, unroll=4)
            def row_body(r):
                splat_r = lanes * jnp.int32(0) + r
                for j in range(4):
                    v = in_v[r, pl.ds(j * 16, 16)]
                    out_v[r, pl.ds(j * 16, 16)] = lax.abs(v)
                for j in range(4):
                    g = [plsc.load_gather(in_v, [splat_r, g3[j][k]])
                         for k in range(3)]
                    acc = (g[0] * g[0] + g[1] * g[1]) + g[2] * g[2]
                    out_v[r, pl.ds(64 + j * 16, 16)] = _sqrt16(acc)
                for j in range(2):
                    g = [plsc.load_gather(in_v, [splat_r, g5[j][k]])
                         for k in range(5)]
                    acc = ((g[0] * g[0] + g[1] * g[1])
                           + (g[2] * g[2] + g[3] * g[3])) + g[4] * g[4]
                    out_v[r, pl.ds(128 + j * 16, 16)] = _sqrt16(acc)

        start_in(in_a, in_sem_a, 0)
        start_in(in_b, in_sem_b, 1)
        wait_in(in_a, in_sem_a)
        compute(in_a, out_a)
        start_out(out_a, out_sem_a, 0)
        start_in(in_a, in_sem_a, 2)
        wait_in(in_b, in_sem_b)
        compute(in_b, out_b)
        start_out(out_b, out_sem_b, 1)

        def chunk_pair(h, carry):
            i0 = h * 2
            start_in(in_b, in_sem_b, i0 + 1)
            wait_in(in_a, in_sem_a)
            wait_out(out_a, out_sem_a)
            compute(in_a, out_a)
            start_out(out_a, out_sem_a, i0)
            start_in(in_a, in_sem_a, i0 + 2)
            wait_in(in_b, in_sem_b)
            wait_out(out_b, out_sem_b)
            compute(in_b, out_b)
            start_out(out_b, out_sem_b, i0 + 1)
            return carry

        if niter % 2:
            lax.fori_loop(1, (niter - 1) // 2, chunk_pair, 0)
            wait_in(in_a, in_sem_a)
            wait_out(out_a, out_sem_a)
            compute(in_a, out_a)
            start_out(out_a, out_sem_a, niter - 1)
        else:
            lax.fori_loop(1, niter // 2, chunk_pair, 0)
            wait_in(in_a, in_sem_a)
        wait_out(out_a, out_sem_a)
        wait_out(out_b, out_sem_b)

    return norm_kernel


_norm_kernels = [_make_norm_kernel(SLAB, r0)
                 for r0 in range(0, N_NODES, SLAB)]


def kernel(features):
    outs = [k(features) for k in _norm_kernels]
    return jnp.concatenate(outs, axis=0)


# speedup vs baseline: 2.2255x; 2.0389x over previous
import functools

import jax
import jax.numpy as jnp
from jax import lax
from jax.experimental import pallas as pl
from jax.experimental.pallas import tpu as pltpu
from jax.experimental.pallas import tpu_sc as plsc

N_NODES = 100000
DIM = 416
NOUT = 160
NC = 2
NS = 16
NW = NC * NS
R = 40
SLAB = 50000


def _sqrt16(x):
    i = lax.bitcast_convert_type(x, jnp.int32)
    i = jnp.int32(0x5F3759DF) - lax.shift_right_arithmetic(i, 1)
    y = lax.bitcast_convert_type(i, jnp.float32)
    xh = x * jnp.float32(0.5)
    y = y * (jnp.float32(1.5) - xh * y * y)
    return x * y


def _make_norm_kernel(n_rows, row0):
    nchunk = n_rows // R
    chunk0 = row0 // R
    niter = -(-nchunk // NW)
    assert n_rows % R == 0 and row0 % R == 0 and niter >= 2
    mesh = plsc.VectorSubcoreMesh(core_axis_name="c", subcore_axis_name="s")

    @functools.partial(
        pl.kernel,
        mesh=mesh,
        out_type=jax.ShapeDtypeStruct((n_rows, NOUT), jnp.float32),
        compiler_params=pltpu.CompilerParams(needs_layout_passes=False),
        scratch_types=[
            pltpu.VMEM((R, DIM), jnp.float32),
            pltpu.VMEM((R, DIM), jnp.float32),
            pltpu.VMEM((R, NOUT), jnp.float32),
            pltpu.VMEM((R, NOUT), jnp.float32),
            pltpu.SemaphoreType.DMA,
            pltpu.SemaphoreType.DMA,
            pltpu.SemaphoreType.DMA,
            pltpu.SemaphoreType.DMA,
        ],
    )
    def norm_kernel(feat_hbm, out_hbm, in_a, in_b, out_a, out_b,
                    in_sem_a, in_sem_b, out_sem_a, out_sem_b):
        wid = lax.axis_index("s") * NC + lax.axis_index("c")
        lanes = lax.iota(jnp.int32, 16)
        l3 = [lanes * jnp.int32(3) + jnp.int32(k) for k in range(3)]
        l5 = [lanes * jnp.int32(5) + jnp.int32(k) for k in range(5)]

        def chunk_of(i):
            return jnp.minimum(i * NW + wid, nchunk - 1)

        def start_in(buf, sem, i):
            r0 = (chunk0 + chunk_of(i)) * R
            pltpu.async_copy(feat_hbm.at[pl.ds(r0, R), :], buf, sem)

        def wait_in(buf, sem):
            pltpu.make_async_copy(feat_hbm.at[pl.ds(0, R), :], buf, sem).wait()

        def start_out(buf, sem, i):
            r0 = chunk_of(i) * R
            pltpu.async_copy(buf, out_hbm.at[pl.ds(r0, R), :], sem)

        def wait_out(buf, sem):
            pltpu.make_async_copy(buf, out_hbm.at[pl.ds(0, R), :], sem).wait()

        g3 = [[l3[k] + jnp.int32(64 + 48 * j) for k in range(3)]
              for j in range(4)]
        g5 = [[l5[k] + jnp.int32(256 + 80 * j) for k in range(5)]
              for j in range(2)]

        def compute(in_v, out_v):
            @plsc.parallel_loop(0, R, unroll=4)
            def row_body(r):
                splat_r = lanes * jnp.int32(0) + r
                for j in range(4):
                    v = in_v[r, pl.ds(j * 16, 16)]
                    out_v[r, pl.ds(j * 16, 16)] = lax.abs(v)
                for j in range(4):
                    g = [plsc.load_gather(in_v, [splat_r, g3[j][k]])
                         for k in range(3)]
                    acc = (g[0] * g[0] + g[1] * g[1]) + g[2] * g[2]
                    out_v[r, pl.ds(64 + j * 16, 16)] = _sqrt16(acc)
                for j in range(2):
                    g = [plsc.load_gather(in_v, [splat_r, g5[j][k]])
                         for k in range(5)]
                    acc = ((g[0] * g[0] + g[1] * g[1])
                           + (g[2] * g[2] + g[3] * g[3])) + g[4] * g[4]
                    out_v[r, pl.ds(128 + j * 16, 16)] = _sqrt16(acc)

        start_in(in_a, in_sem_a, 0)
        start_in(in_b, in_sem_b, 1)
        wait_in(in_a, in_sem_a)
        compute(in_a, out_a)
        start_out(out_a, out_sem_a, 0)
        start_in(in_a, in_sem_a, 2)
        wait_in(in_b, in_sem_b)
        compute(in_b, out_b)
        start_out(out_b, out_sem_b, 1)

        def chunk_pair(h, carry):
            i0 = h * 2
            start_in(in_b, in_sem_b, i0 + 1)
            wait_in(in_a, in_sem_a)
            wait_out(out_a, out_sem_a)
            compute(in_a, out_a)
            start_out(out_a, out_sem_a, i0)
            start_in(in_a, in_sem_a, i0 + 2)
            wait_in(in_b, in_sem_b)
            wait_out(out_b, out_sem_b)
            compute(in_b, out_b)
            start_out(out_b, out_sem_b, i0 + 1)
            return carry

        if niter % 2:
            lax.fori_loop(1, (niter - 1) // 2, chunk_pair, 0)
            wait_in(in_a, in_sem_a)
            wait_out(out_a, out_sem_a)
            compute(in_a, out_a)
            start_out(out_a, out_sem_a, niter - 1)
        else:
            lax.fori_loop(1, niter // 2, chunk_pair, 0)
            wait_in(in_a, in_sem_a)
        wait_out(out_a, out_sem_a)
        wait_out(out_b, out_sem_b)

    return norm_kernel


_norm_kernel = _make_norm_kernel(N_NODES, 0)


def kernel(features):
    return _norm_kernel(features)


# speedup vs baseline: 2.2322x; 1.0030x over previous
import functools

import jax
import jax.numpy as jnp
from jax import lax
from jax.experimental import pallas as pl
from jax.experimental.pallas import tpu as pltpu
from jax.experimental.pallas import tpu_sc as plsc

N_NODES = 100000
DIM = 416
NOUT = 160
NC = 2
NS = 16
NW = NC * NS
R = 80
SLAB = 50000


def _sqrt16(x):
    i = lax.bitcast_convert_type(x, jnp.int32)
    i = jnp.int32(0x5F3759DF) - lax.shift_right_arithmetic(i, 1)
    y = lax.bitcast_convert_type(i, jnp.float32)
    xh = x * jnp.float32(0.5)
    y = y * (jnp.float32(1.5) - xh * y * y)
    return x * y


def _make_norm_kernel(n_rows, row0):
    nchunk = n_rows // R
    chunk0 = row0 // R
    niter = -(-nchunk // NW)
    assert n_rows % R == 0 and row0 % R == 0 and niter >= 2
    mesh = plsc.VectorSubcoreMesh(core_axis_name="c", subcore_axis_name="s")

    @functools.partial(
        pl.kernel,
        mesh=mesh,
        out_type=jax.ShapeDtypeStruct((n_rows, NOUT), jnp.float32),
        compiler_params=pltpu.CompilerParams(needs_layout_passes=False),
        scratch_types=[
            pltpu.VMEM((R, DIM), jnp.float32),
            pltpu.VMEM((R, DIM), jnp.float32),
            pltpu.VMEM((R, NOUT), jnp.float32),
            pltpu.VMEM((R, NOUT), jnp.float32),
            pltpu.SemaphoreType.DMA,
            pltpu.SemaphoreType.DMA,
            pltpu.SemaphoreType.DMA,
            pltpu.SemaphoreType.DMA,
        ],
    )
    def norm_kernel(feat_hbm, out_hbm, in_a, in_b, out_a, out_b,
                    in_sem_a, in_sem_b, out_sem_a, out_sem_b):
        wid = lax.axis_index("s") * NC + lax.axis_index("c")
        lanes = lax.iota(jnp.int32, 16)
        l3 = [lanes * jnp.int32(3) + jnp.int32(k) for k in range(3)]
        l5 = [lanes * jnp.int32(5) + jnp.int32(k) for k in range(5)]

        def chunk_of(i):
            return jnp.minimum(i * NW + wid, nchunk - 1)

        def start_in(buf, sem, i):
            r0 = (chunk0 + chunk_of(i)) * R
            pltpu.async_copy(feat_hbm.at[pl.ds(r0, R), :], buf, sem)

        def wait_in(buf, sem):
            pltpu.make_async_copy(feat_hbm.at[pl.ds(0, R), :], buf, sem).wait()

        def start_out(buf, sem, i):
            r0 = chunk_of(i) * R
            pltpu.async_copy(buf, out_hbm.at[pl.ds(r0, R), :], sem)

        def wait_out(buf, sem):
            pltpu.make_async_copy(buf, out_hbm.at[pl.ds(0, R), :], sem).wait()

        g3 = [[l3[k] + jnp.int32(64 + 48 * j) for k in range(3)]
              for j in range(4)]
        g5 = [[l5[k] + jnp.int32(256 + 80 * j) for k in range(5)]
              for j in range(2)]

        def compute(in_v, out_v):
            @plsc.parallel_loop(0, R, unroll=4)
            def row_body(r):
                splat_r = lanes * jnp.int32(0) + r
                for j in range(4):
                    v = in_v[r, pl.ds(j * 16, 16)]
                    out_v[r, pl.ds(j * 16, 16)] = lax.abs(v)
                for j in range(4):
                    g = [plsc.load_gather(in_v, [splat_r, g3[j][k]])
                         for k in range(3)]
                    acc = (g[0] * g[0] + g[1] * g[1]) + g[2] * g[2]
                    out_v[r, pl.ds(64 + j * 16, 16)] = _sqrt16(acc)
                for j in range(2):
                    g = [plsc.load_gather(in_v, [splat_r, g5[j][k]])
                         for k in range(5)]
                    acc = ((g[0] * g[0] + g[1] * g[1])
                           + (g[2] * g[2] + g[3] * g[3])) + g[4] * g[4]
                    out_v[r, pl.ds(128 + j * 16, 16)] = _sqrt16(acc)

        start_in(in_a, in_sem_a, 0)
        start_in(in_b, in_sem_b, 1)
        wait_in(in_a, in_sem_a)
        compute(in_a, out_a)
        start_out(out_a, out_sem_a, 0)
        start_in(in_a, in_sem_a, 2)
        wait_in(in_b, in_sem_b)
        compute(in_b, out_b)
        start_out(out_b, out_sem_b, 1)

        def chunk_pair(h, carry):
            i0 = h * 2
            start_in(in_b, in_sem_b, i0 + 1)
            wait_in(in_a, in_sem_a)
            wait_out(out_a, out_sem_a)
            compute(in_a, out_a)
            start_out(out_a, out_sem_a, i0)
            start_in(in_a, in_sem_a, i0 + 2)
            wait_in(in_b, in_sem_b)
            wait_out(out_b, out_sem_b)
            compute(in_b, out_b)
            start_out(out_b, out_sem_b, i0 + 1)
            return carry

        if niter % 2:
            lax.fori_loop(1, (niter - 1) // 2, chunk_pair, 0)
            wait_in(in_a, in_sem_a)
            wait_out(out_a, out_sem_a)
            compute(in_a, out_a)
            start_out(out_a, out_sem_a, niter - 1)
        else:
            lax.fori_loop(1, niter // 2, chunk_pair, 0)
            wait_in(in_a, in_sem_a)
        wait_out(out_a, out_sem_a)
        wait_out(out_b, out_sem_b)

    return norm_kernel


_norm_kernel = _make_norm_kernel(N_NODES, 0)


def kernel(features):
    return _norm_kernel(features)
